# baseline (speedup 1.0000x reference)
import jax
import jax.numpy as jnp
from jax import lax
from jax.experimental import pallas as pl
from jax.experimental.pallas import tpu as pltpu
from jax.experimental.pallas import tpu_sc as plsc

B, C, H, W = 4, 384, 224, 224
P = B * H * W
NG = C // 16

_info = plsc.get_sparse_core_info()
_NC = _info.num_cores
_NS = _info.num_subcores
NW = _NC * _NS
PPW = P // NW
PCH = 64
NCHUNK = PPW // PCH


def _sc_body(in_hbm, out_hbm, bin0, bin1, bout0, bout1,
             gsem0, gsem1, ssem0, ssem1):
    bins = (bin0, bin1)
    bouts = (bout0, bout1)
    gsem = (gsem0, gsem1)
    ssem = (ssem0, ssem1)

    wid = lax.axis_index("s") * _NC + lax.axis_index("c")
    pix0 = wid * PPW

    def gather(t, slot):
        pltpu.async_copy(in_hbm.at[pl.ds(pix0 + t * PCH, PCH)], bins[slot],
                         gsem[slot])

    gather(0, 0)
    gather(1, 1)

    @pl.loop(0, NCHUNK, step=2)
    def _(t0):
        for slot in range(2):
            t = t0 + slot
            src = in_hbm.at[pl.ds(pix0 + t * PCH, PCH)]
            dst = out_hbm.at[pl.ds(pix0 + t * PCH, PCH)]
            pltpu.make_async_copy(src, bins[slot], gsem[slot]).wait()

            @pl.when(t >= 2)
            def _():
                pltpu.make_async_copy(bouts[slot], dst, ssem[slot]).wait()

            @pl.loop(0, PCH, unroll=4)
            def _(p):
                for j in range(NG):
                    v = bins[slot][p, pl.ds(16 * (NG - 1 - j), 16)]
                    bouts[slot][p, pl.ds(16 * j, 16)] = lax.rev(v, (0,))

            pltpu.async_copy(bouts[slot], dst, ssem[slot])

            @pl.when(t + 2 < NCHUNK)
            def _():
                gather(t + 2, slot)

    for slot in range(2):
        t = NCHUNK - 2 + slot
        dst = out_hbm.at[pl.ds(pix0 + t * PCH, PCH)]
        pltpu.make_async_copy(bouts[slot], dst, ssem[slot]).wait()


_sc_kernel = pl.kernel(
    _sc_body,
    out_type=jax.ShapeDtypeStruct((P, C), jnp.float32),
    mesh=plsc.VectorSubcoreMesh(core_axis_name="c", subcore_axis_name="s"),
    scratch_types=[
        pltpu.VMEM((PCH, C), jnp.float32),
        pltpu.VMEM((PCH, C), jnp.float32),
        pltpu.VMEM((PCH, C), jnp.float32),
        pltpu.VMEM((PCH, C), jnp.float32),
        pltpu.SemaphoreType.DMA,
        pltpu.SemaphoreType.DMA,
        pltpu.SemaphoreType.DMA,
        pltpu.SemaphoreType.DMA,
    ],
)


@jax.jit
def kernel(input):
    xt = jnp.transpose(input, (0, 2, 3, 1)).reshape(P, C)
    yt = _sc_kernel(xt)
    return jnp.transpose(yt.reshape(B, H, W, C), (0, 3, 1, 2))


# speedup vs baseline: 2.3551x; 2.3551x over previous
import jax
import jax.numpy as jnp
from jax import lax
from jax.experimental import pallas as pl
from jax.experimental.pallas import tpu as pltpu
from jax.experimental.pallas import tpu_sc as plsc

B, C, H, W = 4, 384, 224, 224
P = B * H * W
NG = C // 16

_info = plsc.get_sparse_core_info()
_NC = _info.num_cores
_NS = _info.num_subcores
NW = _NC * _NS
PPW = P // NW
PCH = 32
NCHUNK = PPW // PCH
NSLOT = 4


def _sc_body(in_hbm, out_hbm, *rest):
    bins = rest[:NSLOT]
    bouts = rest[NSLOT:2 * NSLOT]
    gsem = rest[2 * NSLOT:3 * NSLOT]
    ssem = rest[3 * NSLOT:4 * NSLOT]

    wid = lax.axis_index("s") * _NC + lax.axis_index("c")
    pix0 = wid * PPW

    def gather(t, slot):
        pltpu.async_copy(in_hbm.at[pl.ds(pix0 + t * PCH, PCH)], bins[slot],
                         gsem[slot])

    for k in range(NSLOT):
        gather(k, k)

    @pl.loop(0, NCHUNK, step=NSLOT)
    def _(t0):
        for slot in range(NSLOT):
            t = t0 + slot
            src = in_hbm.at[pl.ds(pix0 + t * PCH, PCH)]
            dst = out_hbm.at[pl.ds(pix0 + t * PCH, PCH)]
            pltpu.make_async_copy(src, bins[slot], gsem[slot]).wait()

            @pl.when(t >= NSLOT)
            def _():
                pltpu.make_async_copy(bouts[slot], dst, ssem[slot]).wait()

            @pl.loop(0, PCH)
            def _(p):
                for j in range(NG):
                    v = bins[slot][p, pl.ds(16 * (NG - 1 - j), 16)]
                    bouts[slot][p, pl.ds(16 * j, 16)] = lax.rev(v, (0,))

            pltpu.async_copy(bouts[slot], dst, ssem[slot])

            @pl.when(t + NSLOT < NCHUNK)
            def _():
                gather(t + NSLOT, slot)

    for slot in range(NSLOT):
        t = NCHUNK - NSLOT + slot
        dst = out_hbm.at[pl.ds(pix0 + t * PCH, PCH)]
        pltpu.make_async_copy(bouts[slot], dst, ssem[slot]).wait()


_sc_kernel = pl.kernel(
    _sc_body,
    out_type=jax.ShapeDtypeStruct((P, C), jnp.float32),
    mesh=plsc.VectorSubcoreMesh(core_axis_name="c", subcore_axis_name="s"),
    scratch_types=(
        [pltpu.VMEM((PCH, C), jnp.float32) for _ in range(2 * NSLOT)]
        + [pltpu.SemaphoreType.DMA for _ in range(2 * NSLOT)]
    ),
)


@jax.jit
def kernel(input):
    xt = jnp.transpose(input, (0, 2, 3, 1)).reshape(P, C)
    yt = _sc_kernel(xt)
    return jnp.transpose(yt.reshape(B, H, W, C), (0, 3, 1, 2))


# speedup vs baseline: 2.3596x; 1.0019x over previous
import jax
import jax.numpy as jnp
from jax import lax
from jax.experimental import pallas as pl
from jax.experimental.pallas import tpu as pltpu
from jax.experimental.pallas import tpu_sc as plsc

B, C, H, W = 4, 384, 224, 224
P = B * H * W
NG = C // 16

_info = plsc.get_sparse_core_info()
_NC = _info.num_cores
_NS = _info.num_subcores
NW = _NC * _NS
PPW = P // NW
PCH = 32
NCHUNK = PPW // PCH
NSLOT = 4


def _sc_body(in_hbm, out_hbm, *rest):
    bins = rest[:NSLOT]
    bouts = rest[NSLOT:2 * NSLOT]
    gsem = rest[2 * NSLOT:3 * NSLOT]
    ssem = rest[3 * NSLOT:4 * NSLOT]

    wid = lax.axis_index("s") * _NC + lax.axis_index("c")
    pix0 = wid * PPW

    def gather(t, slot):
        pltpu.async_copy(in_hbm.at[pl.ds(pix0 + t * PCH, PCH)], bins[slot],
                         gsem[slot])

    for k in range(NSLOT):
        gather(k, k)

    @pl.loop(0, NCHUNK, step=NSLOT)
    def _(t0):
        for slot in range(NSLOT):
            t = t0 + slot
            src = in_hbm.at[pl.ds(pix0 + t * PCH, PCH)]
            dst = out_hbm.at[pl.ds(pix0 + t * PCH, PCH)]
            pltpu.make_async_copy(src, bins[slot], gsem[slot]).wait()

            @pl.when(t >= NSLOT)
            def _():
                pltpu.make_async_copy(bouts[slot], dst, ssem[slot]).wait()

            @plsc.parallel_loop(0, PCH)
            def _(p):
                for j in range(NG):
                    v = bins[slot][p, pl.ds(16 * (NG - 1 - j), 16)]
                    bouts[slot][p, pl.ds(16 * j, 16)] = lax.rev(v, (0,))

            pltpu.async_copy(bouts[slot], dst, ssem[slot])

            @pl.when(t + NSLOT < NCHUNK)
            def _():
                gather(t + NSLOT, slot)

    for slot in range(NSLOT):
        t = NCHUNK - NSLOT + slot
        dst = out_hbm.at[pl.ds(pix0 + t * PCH, PCH)]
        pltpu.make_async_copy(bouts[slot], dst, ssem[slot]).wait()


_sc_kernel = pl.kernel(
    _sc_body,
    out_type=jax.ShapeDtypeStruct((P, C), jnp.float32),
    mesh=plsc.VectorSubcoreMesh(core_axis_name="c", subcore_axis_name="s"),
    scratch_types=(
        [pltpu.VMEM((PCH, C), jnp.float32) for _ in range(2 * NSLOT)]
        + [pltpu.SemaphoreType.DMA for _ in range(2 * NSLOT)]
    ),
)


@jax.jit
def kernel(input):
    xt = jnp.transpose(input, (0, 2, 3, 1)).reshape(P, C)
    yt = _sc_kernel(xt)
    return jnp.transpose(yt.reshape(B, H, W, C), (0, 3, 1, 2))
